# baseline (speedup 1.0000x reference)
import functools

import jax
import jax.numpy as jnp
from jax import lax
from jax.experimental import pallas as pl
from jax.experimental.pallas import tpu as pltpu
from jax.experimental.pallas import tpu_sc as plsc

N = 10000
E = 320000
D_IN = 128
D_HID = 128
C = 64
K = 10
ALPHA = 0.1

NH = 5000
HALF = 5120
NPAD = 2 * HALF
RPT = HALF // 16
CH = 512
CPT = 20
UB = 128
TPE = CH * CPT
EHP = 16 * TPE


def _idx_prep_body(row_ref, col_ref, rowm_ref, c0_ref, c1_ref):
    r = row_ref[...]
    c = col_ref[...]
    io8 = lax.broadcasted_iota(jnp.int32, r.shape, 1) & 7
    dummy = NH + io8
    rowm_ref[...] = jnp.where(r >= NH, r + (HALF - NH), r)
    c0_ref[...] = jnp.where(c < NH, c, dummy)
    c1_ref[...] = jnp.where(c >= NH, c - NH, dummy)


def _idx_prep(row2d, col2d):
    shp = jax.ShapeDtypeStruct(row2d.shape, jnp.int32)
    return pl.pallas_call(
        _idx_prep_body,
        out_shape=[shp, shp, shp],
    )(row2d, col2d)


def _mlp_body(x_ref, w1_ref, b1_ref, w2_ref, b2_ref, o_ref):
    h = jnp.dot(x_ref[...], w1_ref[...], preferred_element_type=jnp.float32)
    h = jnp.maximum(h + b1_ref[...], 0.0)
    o = jnp.dot(h, w2_ref[...], preferred_element_type=jnp.float32)
    o_ref[...] = o + b2_ref[...]


def _mlp(x, W1, b1, W2, b2):
    return pl.pallas_call(
        _mlp_body,
        grid=(10,),
        in_specs=[
            pl.BlockSpec((N // 10, D_IN), lambda i: (i, 0)),
            pl.BlockSpec((D_IN, D_HID), lambda i: (0, 0)),
            pl.BlockSpec((1, D_HID), lambda i: (0, 0)),
            pl.BlockSpec((D_HID, C), lambda i: (0, 0)),
            pl.BlockSpec((1, C), lambda i: (0, 0)),
        ],
        out_specs=pl.BlockSpec((N // 10, C), lambda i: (i, 0)),
        out_shape=jax.ShapeDtypeStruct((N, C), jnp.float32),
    )(x, W1, b1.reshape(1, D_HID), W2, b2.reshape(1, C))


def _prep_body(sdeg_ref, z_ref, s0_ref, d2_ref, zz_ref):
    deg = sdeg_ref[...]
    z = z_ref[...]
    rows = lax.broadcasted_iota(jnp.int32, deg.shape, 0)
    m = rows < NH
    dinv = jnp.where(m, lax.rsqrt(jnp.maximum(deg, 1e-12)), 0.0)
    s0_ref[...] = dinv * z
    d2_ref[...] = (1.0 - ALPHA) * dinv * dinv
    zz_ref[...] = ALPHA * dinv * z


def _prep(sdeg, zp):
    shp = jax.ShapeDtypeStruct((NPAD, C), jnp.float32)
    spec = pl.BlockSpec((HALF, C), lambda i: (i, 0))
    return pl.pallas_call(
        _prep_body,
        grid=(2,),
        in_specs=[spec, spec],
        out_specs=[spec, spec, spec],
        out_shape=[shp, shp, shp],
    )(sdeg, zp)


def _final_body(s_ref, sdeg_ref, o_ref):
    deg = jnp.maximum(sdeg_ref[...], 1e-12)
    o = s_ref[...] * jnp.sqrt(deg)
    mx = jnp.max(o, axis=1, keepdims=True)
    e = jnp.exp(o - mx)
    lse = jnp.log(jnp.sum(e, axis=1, keepdims=True))
    o_ref[...] = (o - mx) - lse


def _final(s, sdeg):
    spec = pl.BlockSpec((HALF, C), lambda i: (i, 0))
    return pl.pallas_call(
        _final_body,
        grid=(2,),
        in_specs=[spec, spec],
        out_specs=spec,
        out_shape=jax.ShapeDtypeStruct((NPAD, C), jnp.float32),
    )(s, sdeg)


_MESH = plsc.VectorSubcoreMesh(core_axis_name="c", subcore_axis_name="s")


@functools.partial(
    pl.kernel,
    out_type=jax.ShapeDtypeStruct((NPAD, C), jnp.float32),
    mesh=_MESH,
    compiler_params=pltpu.CompilerParams(use_tc_tiling_on_sc=False),
    scratch_types=[
        pltpu.VMEM_SHARED((HALF, C), jnp.float32),
        pltpu.VMEM((CPT, CH), jnp.int32),
        pltpu.VMEM((CPT, CH), jnp.int32),
        pltpu.VMEM((UB, C), jnp.float32),
        pltpu.VMEM((CH, C), jnp.float32),
        pltpu.VMEM((CH, C), jnp.float32),
        pltpu.VMEM((UB, C), jnp.float32),
        pltpu.SemaphoreType.DMA,
        pltpu.SemaphoreType.DMA,
    ],
)
def _sc_round(s_h, rowp_h, colf_h, d2_h, zz_h, out_h,
              acc, idxr, idxc, g0, g1, g2, g3, semg, sems):
    cid = lax.axis_index("c")
    sid = lax.axis_index("s")
    gb = (g0, g1, g2, g3)

    pltpu.sync_copy(rowp_h.at[cid, sid], idxr)
    pltpu.sync_copy(colf_h.at[cid, sid], idxc)

    zero16 = jnp.zeros((16,), jnp.float32)

    def _z(i, _):
        g0[i >> 2, pl.ds((i & 3) * 16, 16)] = zero16
        return 0

    lax.fori_loop(0, UB * (C // 16), _z, 0)
    abase = sid * RPT
    pltpu.sync_copy(g0, acc.at[pl.ds(abase, UB)])
    pltpu.sync_copy(g0, acc.at[pl.ds(abase + UB, UB)])
    pltpu.sync_copy(g0.at[pl.ds(0, RPT - 2 * UB)],
                    acc.at[pl.ds(abase + 2 * UB, RPT - 2 * UB)])
    plsc.subcore_barrier()

    pltpu.async_copy(s_h.at[idxr.at[0]], g1, semg)

    def _grp(t, _):
        a = 2 * t
        pltpu.make_async_copy(s_h.at[idxr.at[a]], g1, semg).wait()
        pltpu.async_copy(s_h.at[idxr.at[a + 1]], g2, sems)
        pltpu.sync_copy(g1, acc.at[idxc.at[a]], add=True)
        pltpu.make_async_copy(s_h.at[idxr.at[a + 1]], g2, sems).wait()

        @pl.when(a + 2 < CPT)
        def _():
            pltpu.async_copy(s_h.at[idxr.at[a + 2]], g1, semg)

        pltpu.sync_copy(g2, acc.at[idxc.at[a + 1]], add=True)
        return 0

    lax.fori_loop(0, CPT // 2, _grp, 0)
    plsc.subcore_barrier()

    pbase = cid * HALF + sid * RPT
    for off, nn in ((0, UB), (UB, UB), (2 * UB, RPT - 2 * UB)):
        pltpu.sync_copy(acc.at[pl.ds(abase + off, nn)], g0.at[pl.ds(0, nn)])
        pltpu.sync_copy(s_h.at[pl.ds(pbase + off, nn)], g1.at[pl.ds(0, nn)])
        pltpu.sync_copy(d2_h.at[pl.ds(pbase + off, nn)], g2.at[pl.ds(0, nn)])
        pltpu.sync_copy(zz_h.at[pl.ds(pbase + off, nn)], g3.at[pl.ds(0, nn)])

        def _u(i, _):
            r = i >> 2
            lo = (i & 3) * 16
            a = g0[r, pl.ds(lo, 16)]
            sv = g1[r, pl.ds(lo, 16)]
            dv = g2[r, pl.ds(lo, 16)]
            zv = g3[r, pl.ds(lo, 16)]
            g0[r, pl.ds(lo, 16)] = dv * (a + sv) + zv
            return 0

        lax.fori_loop(0, nn * (C // 16), _u, 0)
        pltpu.sync_copy(g0.at[pl.ds(0, nn)], out_h.at[pl.ds(pbase + off, nn)])


def kernel(x, edge_index, W1, b1, W2, b2):
    row2d = edge_index[0].reshape(E // 128, 128)
    col2d = edge_index[1].reshape(E // 128, 128)
    rowm, c0, c1 = _idx_prep(row2d, col2d)

    rowm_e = jnp.concatenate([rowm.reshape(-1),
                              jnp.full((1,), NH, jnp.int32)])
    c0_e = jnp.concatenate([c0.reshape(-1), jnp.full((1,), NH, jnp.int32)])
    c1_e = jnp.concatenate([c1.reshape(-1), jnp.full((1,), NH, jnp.int32)])
    key0 = edge_index[1] < NH
    pos0 = jnp.cumsum(key0.astype(jnp.int32)) - 1
    pos1 = jnp.cumsum(1 - key0.astype(jnp.int32)) - 1
    idxe = jnp.arange(E, dtype=jnp.int32)
    oob = jnp.int32(EHP)
    list0 = jnp.full((EHP,), E, jnp.int32).at[
        jnp.where(key0, pos0, oob)].set(idxe, mode="drop")
    list1 = jnp.full((EHP,), E, jnp.int32).at[
        jnp.where(key0, oob, pos1)].set(idxe, mode="drop")
    rowp = jnp.stack([rowm_e[list0], rowm_e[list1]]).reshape(2, 16, CPT, CH)
    colf = jnp.stack([c0_e[list0], c1_e[list1]]).reshape(2, 16, CPT, CH)

    h = _mlp(x, W1, b1, W2, b2)
    zpad = jnp.zeros((HALF - NH, C), jnp.float32)
    zp = jnp.concatenate([h[:NH], zpad, h[NH:], zpad])

    ones = jnp.ones((NPAD, C), jnp.float32)
    zeros = jnp.zeros((NPAD, C), jnp.float32)
    sdeg = _sc_round(ones, rowp, colf, ones, zeros)

    s, D2, ZZ = _prep(sdeg, zp)
    for _ in range(K):
        s = _sc_round(s, rowp, colf, D2, ZZ)

    o = _final(s, sdeg)
    return jnp.concatenate([o[:NH], o[HALF:HALF + NH]])


# speedup vs baseline: 1.0155x; 1.0155x over previous
import functools

import jax
import jax.numpy as jnp
from jax import lax
from jax.experimental import pallas as pl
from jax.experimental.pallas import tpu as pltpu
from jax.experimental.pallas import tpu_sc as plsc

N = 10000
E = 320000
D_IN = 128
D_HID = 128
C = 64
K = 10
ALPHA = 0.1

NH = 5000
HALF = 5120
NPAD = 2 * HALF
RPT = HALF // 16
CH = 256
CPT = 40
UB = 128
TPE = CH * CPT
EHP = 16 * TPE


def _idx_prep_body(row_ref, col_ref, rowm_ref, c0_ref, c1_ref):
    r = row_ref[...]
    c = col_ref[...]
    io8 = lax.broadcasted_iota(jnp.int32, r.shape, 1) & 7
    dummy = NH + io8
    rowm_ref[...] = jnp.where(r >= NH, r + (HALF - NH), r)
    c0_ref[...] = jnp.where(c < NH, c, dummy)
    c1_ref[...] = jnp.where(c >= NH, c - NH, dummy)


def _idx_prep(row2d, col2d):
    shp = jax.ShapeDtypeStruct(row2d.shape, jnp.int32)
    return pl.pallas_call(
        _idx_prep_body,
        out_shape=[shp, shp, shp],
    )(row2d, col2d)


def _mlp_body(x_ref, w1_ref, b1_ref, w2_ref, b2_ref, o_ref):
    h = jnp.dot(x_ref[...], w1_ref[...], preferred_element_type=jnp.float32)
    h = jnp.maximum(h + b1_ref[...], 0.0)
    o = jnp.dot(h, w2_ref[...], preferred_element_type=jnp.float32)
    o_ref[...] = o + b2_ref[...]


def _mlp(x, W1, b1, W2, b2):
    return pl.pallas_call(
        _mlp_body,
        grid=(10,),
        in_specs=[
            pl.BlockSpec((N // 10, D_IN), lambda i: (i, 0)),
            pl.BlockSpec((D_IN, D_HID), lambda i: (0, 0)),
            pl.BlockSpec((1, D_HID), lambda i: (0, 0)),
            pl.BlockSpec((D_HID, C), lambda i: (0, 0)),
            pl.BlockSpec((1, C), lambda i: (0, 0)),
        ],
        out_specs=pl.BlockSpec((N // 10, C), lambda i: (i, 0)),
        out_shape=jax.ShapeDtypeStruct((N, C), jnp.float32),
    )(x, W1, b1.reshape(1, D_HID), W2, b2.reshape(1, C))


def _prep_body(sdeg_ref, z_ref, s0_ref, d2_ref, zz_ref):
    deg = sdeg_ref[...]
    z = z_ref[...]
    rows = lax.broadcasted_iota(jnp.int32, deg.shape, 0)
    m = rows < NH
    dinv = jnp.where(m, lax.rsqrt(jnp.maximum(deg, 1e-12)), 0.0)
    s0_ref[...] = dinv * z
    d2_ref[...] = (1.0 - ALPHA) * dinv * dinv
    zz_ref[...] = ALPHA * dinv * z


def _prep(sdeg, zp):
    shp = jax.ShapeDtypeStruct((NPAD, C), jnp.float32)
    spec = pl.BlockSpec((HALF, C), lambda i: (i, 0))
    return pl.pallas_call(
        _prep_body,
        grid=(2,),
        in_specs=[spec, spec],
        out_specs=[spec, spec, spec],
        out_shape=[shp, shp, shp],
    )(sdeg, zp)


def _final_body(s_ref, sdeg_ref, o_ref):
    deg = jnp.maximum(sdeg_ref[...], 1e-12)
    o = s_ref[...] * jnp.sqrt(deg)
    mx = jnp.max(o, axis=1, keepdims=True)
    e = jnp.exp(o - mx)
    lse = jnp.log(jnp.sum(e, axis=1, keepdims=True))
    o_ref[...] = (o - mx) - lse


def _final(s, sdeg):
    spec = pl.BlockSpec((HALF, C), lambda i: (i, 0))
    return pl.pallas_call(
        _final_body,
        grid=(2,),
        in_specs=[spec, spec],
        out_specs=spec,
        out_shape=jax.ShapeDtypeStruct((NPAD, C), jnp.float32),
    )(s, sdeg)


_MESH = plsc.VectorSubcoreMesh(core_axis_name="c", subcore_axis_name="s")


@functools.partial(
    pl.kernel,
    out_type=jax.ShapeDtypeStruct((NPAD, C), jnp.float32),
    mesh=_MESH,
    compiler_params=pltpu.CompilerParams(use_tc_tiling_on_sc=False),
    scratch_types=[
        pltpu.VMEM_SHARED((HALF, C), jnp.float32),
        pltpu.VMEM((CPT, CH), jnp.int32),
        pltpu.VMEM((CPT, CH), jnp.int32),
        pltpu.VMEM((CH, C), jnp.float32),
        pltpu.VMEM((CH, C), jnp.float32),
        pltpu.VMEM((CH, C), jnp.float32),
        pltpu.VMEM((CH, C), jnp.float32),
        pltpu.SemaphoreType.DMA,
        pltpu.SemaphoreType.DMA,
        pltpu.SemaphoreType.DMA,
        pltpu.SemaphoreType.DMA,
    ],
)
def _sc_round(s_h, rowp_h, colf_h, d2_h, zz_h, out_h,
              acc, idxr, idxc, g0, g1, g2, g3, sem0, sem1, sem2, sem3):
    cid = lax.axis_index("c")
    sid = lax.axis_index("s")
    gb = (g0, g1, g2, g3)

    pltpu.sync_copy(rowp_h.at[cid, sid], idxr)
    pltpu.sync_copy(colf_h.at[cid, sid], idxc)

    zero16 = jnp.zeros((16,), jnp.float32)

    def _z(i, _):
        g0[i >> 2, pl.ds((i & 3) * 16, 16)] = zero16
        return 0

    lax.fori_loop(0, UB * (C // 16), _z, 0)
    abase = sid * RPT
    pltpu.sync_copy(g0.at[pl.ds(0, UB)], acc.at[pl.ds(abase, UB)])
    pltpu.sync_copy(g0.at[pl.ds(0, UB)], acc.at[pl.ds(abase + UB, UB)])
    pltpu.sync_copy(g0.at[pl.ds(0, RPT - 2 * UB)],
                    acc.at[pl.ds(abase + 2 * UB, RPT - 2 * UB)])
    plsc.subcore_barrier()

    bufs = (g0, g1, g2, g3)
    sems = (sem0, sem1, sem2, sem3)
    for b in range(4):
        pltpu.async_copy(s_h.at[idxr.at[b]], bufs[b], sems[b])

    def _grp(t, _):
        for b in range(4):
            k = 4 * t + b
            pltpu.make_async_copy(s_h.at[idxr.at[k]], bufs[b], sems[b]).wait()
            pltpu.sync_copy(bufs[b], acc.at[idxc.at[k]], add=True)

            @pl.when(k + 4 < CPT)
            def _():
                pltpu.async_copy(s_h.at[idxr.at[k + 4]], bufs[b], sems[b])

        return 0

    lax.fori_loop(0, CPT // 4, _grp, 0)
    plsc.subcore_barrier()

    pbase = cid * HALF + sid * RPT
    for off, nn in ((0, UB), (UB, UB), (2 * UB, RPT - 2 * UB)):
        pltpu.sync_copy(acc.at[pl.ds(abase + off, nn)], g0.at[pl.ds(0, nn)])
        pltpu.sync_copy(s_h.at[pl.ds(pbase + off, nn)], g1.at[pl.ds(0, nn)])
        pltpu.sync_copy(d2_h.at[pl.ds(pbase + off, nn)], g2.at[pl.ds(0, nn)])
        pltpu.sync_copy(zz_h.at[pl.ds(pbase + off, nn)], g3.at[pl.ds(0, nn)])

        def _u(i, _):
            r = i >> 2
            lo = (i & 3) * 16
            a = g0[r, pl.ds(lo, 16)]
            sv = g1[r, pl.ds(lo, 16)]
            dv = g2[r, pl.ds(lo, 16)]
            zv = g3[r, pl.ds(lo, 16)]
            g0[r, pl.ds(lo, 16)] = dv * (a + sv) + zv
            return 0

        lax.fori_loop(0, nn * (C // 16), _u, 0)
        pltpu.sync_copy(g0.at[pl.ds(0, nn)], out_h.at[pl.ds(pbase + off, nn)])


def kernel(x, edge_index, W1, b1, W2, b2):
    row2d = edge_index[0].reshape(E // 128, 128)
    col2d = edge_index[1].reshape(E // 128, 128)
    rowm, c0, c1 = _idx_prep(row2d, col2d)

    rowm_e = jnp.concatenate([rowm.reshape(-1),
                              jnp.full((1,), NH, jnp.int32)])
    c0_e = jnp.concatenate([c0.reshape(-1), jnp.full((1,), NH, jnp.int32)])
    c1_e = jnp.concatenate([c1.reshape(-1), jnp.full((1,), NH, jnp.int32)])
    key0 = edge_index[1] < NH
    pos0 = jnp.cumsum(key0.astype(jnp.int32)) - 1
    pos1 = jnp.cumsum(1 - key0.astype(jnp.int32)) - 1
    idxe = jnp.arange(E, dtype=jnp.int32)
    oob = jnp.int32(EHP)
    list0 = jnp.full((EHP,), E, jnp.int32).at[
        jnp.where(key0, pos0, oob)].set(idxe, mode="drop")
    list1 = jnp.full((EHP,), E, jnp.int32).at[
        jnp.where(key0, oob, pos1)].set(idxe, mode="drop")
    rowp = jnp.stack([rowm_e[list0], rowm_e[list1]]).reshape(2, 16, CPT, CH)
    colf = jnp.stack([c0_e[list0], c1_e[list1]]).reshape(2, 16, CPT, CH)

    h = _mlp(x, W1, b1, W2, b2)
    zpad = jnp.zeros((HALF - NH, C), jnp.float32)
    zp = jnp.concatenate([h[:NH], zpad, h[NH:], zpad])

    ones = jnp.ones((NPAD, C), jnp.float32)
    zeros = jnp.zeros((NPAD, C), jnp.float32)
    sdeg = _sc_round(ones, rowp, colf, ones, zeros)

    s, D2, ZZ = _prep(sdeg, zp)
    for _ in range(K):
        s = _sc_round(s, rowp, colf, D2, ZZ)

    o = _final(s, sdeg)
    return jnp.concatenate([o[:NH], o[HALF:HALF + NH]])


# speedup vs baseline: 1.0293x; 1.0136x over previous
import functools

import jax
import jax.numpy as jnp
from jax import lax
from jax.experimental import pallas as pl
from jax.experimental.pallas import tpu as pltpu
from jax.experimental.pallas import tpu_sc as plsc

N = 10000
E = 320000
D_IN = 128
D_HID = 128
C = 64
K = 10
ALPHA = 0.1

NH = 5000
HALF = 5120
NPAD = 2 * HALF
RPT = HALF // 16
CH = 256
CPT = 40
UB = 128
TPE = CH * CPT
EHP = 16 * TPE


def _idx_prep_body(row_ref, col_ref, rowm_ref, c0_ref, c1_ref):
    r = row_ref[...]
    c = col_ref[...]
    io8 = lax.broadcasted_iota(jnp.int32, r.shape, 1) & 7
    dummy = NH + io8
    rowm_ref[...] = jnp.where(r >= NH, r + (HALF - NH), r)
    c0_ref[...] = jnp.where(c < NH, c, dummy)
    c1_ref[...] = jnp.where(c >= NH, c - NH, dummy)


def _idx_prep(row2d, col2d):
    shp = jax.ShapeDtypeStruct(row2d.shape, jnp.int32)
    return pl.pallas_call(
        _idx_prep_body,
        out_shape=[shp, shp, shp],
    )(row2d, col2d)


def _mlp_body(x_ref, w1_ref, b1_ref, w2_ref, b2_ref, o_ref):
    h = jnp.dot(x_ref[...], w1_ref[...], preferred_element_type=jnp.float32)
    h = jnp.maximum(h + b1_ref[...], 0.0)
    o = jnp.dot(h, w2_ref[...], preferred_element_type=jnp.float32)
    o_ref[...] = o + b2_ref[...]


def _mlp(x, W1, b1, W2, b2):
    return pl.pallas_call(
        _mlp_body,
        grid=(10,),
        in_specs=[
            pl.BlockSpec((N // 10, D_IN), lambda i: (i, 0)),
            pl.BlockSpec((D_IN, D_HID), lambda i: (0, 0)),
            pl.BlockSpec((1, D_HID), lambda i: (0, 0)),
            pl.BlockSpec((D_HID, C), lambda i: (0, 0)),
            pl.BlockSpec((1, C), lambda i: (0, 0)),
        ],
        out_specs=pl.BlockSpec((N // 10, C), lambda i: (i, 0)),
        out_shape=jax.ShapeDtypeStruct((N, C), jnp.float32),
    )(x, W1, b1.reshape(1, D_HID), W2, b2.reshape(1, C))


def _prep_body(sdeg_ref, z_ref, s0_ref, d2_ref, zz_ref):
    deg = sdeg_ref[...]
    z = z_ref[...]
    rows = lax.broadcasted_iota(jnp.int32, deg.shape, 0)
    m = rows < NH
    dinv = jnp.where(m, lax.rsqrt(jnp.maximum(deg, 1e-12)), 0.0)
    s0_ref[...] = dinv * z
    d2_ref[...] = (1.0 - ALPHA) * dinv * dinv
    zz_ref[...] = ALPHA * dinv * z


def _prep(sdeg, zp):
    shp = jax.ShapeDtypeStruct((NPAD, C), jnp.float32)
    spec = pl.BlockSpec((HALF, C), lambda i: (i, 0))
    return pl.pallas_call(
        _prep_body,
        grid=(2,),
        in_specs=[spec, spec],
        out_specs=[spec, spec, spec],
        out_shape=[shp, shp, shp],
    )(sdeg, zp)


def _final_body(s_ref, sdeg_ref, o_ref):
    deg = jnp.maximum(sdeg_ref[...], 1e-12)
    o = s_ref[...] * jnp.sqrt(deg)
    mx = jnp.max(o, axis=1, keepdims=True)
    e = jnp.exp(o - mx)
    lse = jnp.log(jnp.sum(e, axis=1, keepdims=True))
    o_ref[...] = (o - mx) - lse


def _final(s, sdeg):
    spec = pl.BlockSpec((HALF, C), lambda i: (i, 0))
    return pl.pallas_call(
        _final_body,
        grid=(2,),
        in_specs=[spec, spec],
        out_specs=spec,
        out_shape=jax.ShapeDtypeStruct((NPAD, C), jnp.float32),
    )(s, sdeg)


_MESH = plsc.VectorSubcoreMesh(core_axis_name="c", subcore_axis_name="s")


@functools.partial(
    pl.kernel,
    out_type=jax.ShapeDtypeStruct((NPAD, C), jnp.float32),
    mesh=_MESH,
    compiler_params=pltpu.CompilerParams(use_tc_tiling_on_sc=False),
    scratch_types=[
        pltpu.VMEM_SHARED((HALF, C), jnp.float32),
        pltpu.VMEM((CPT, CH), jnp.int32),
        pltpu.VMEM((CPT, CH), jnp.int32),
        pltpu.VMEM((CH, C), jnp.float32),
        pltpu.VMEM((CH, C), jnp.float32),
        pltpu.VMEM((CH, C), jnp.float32),
        pltpu.VMEM((CH, C), jnp.float32),
        pltpu.SemaphoreType.DMA,
        pltpu.SemaphoreType.DMA,
        pltpu.SemaphoreType.DMA,
        pltpu.SemaphoreType.DMA,
    ],
)
def _sc_round(s_h, rowp_h, colf_h, d2_h, zz_h, out_h,
              acc, idxr, idxc, g0, g1, g2, g3, sem0, sem1, sem2, sem3):
    cid = lax.axis_index("c")
    sid = lax.axis_index("s")
    gb = (g0, g1, g2, g3)

    pltpu.sync_copy(rowp_h.at[cid, sid], idxr)
    pltpu.sync_copy(colf_h.at[cid, sid], idxc)

    zero16 = jnp.zeros((16,), jnp.float32)

    def _z(i, _):
        g0[i >> 2, pl.ds((i & 3) * 16, 16)] = zero16
        return 0

    lax.fori_loop(0, UB * (C // 16), _z, 0)
    abase = sid * RPT
    pltpu.sync_copy(g0.at[pl.ds(0, UB)], acc.at[pl.ds(abase, UB)])
    pltpu.sync_copy(g0.at[pl.ds(0, UB)], acc.at[pl.ds(abase + UB, UB)])
    pltpu.sync_copy(g0.at[pl.ds(0, RPT - 2 * UB)],
                    acc.at[pl.ds(abase + 2 * UB, RPT - 2 * UB)])
    plsc.subcore_barrier()

    bufs = (g0, g1, g2, g3)
    sems = (sem0, sem1, sem2, sem3)
    for b in range(4):
        pltpu.async_copy(s_h.at[idxr.at[b]], bufs[b], sems[b])

    def _grp(t, _):
        for b in range(4):
            k = 4 * t + b
            pltpu.make_async_copy(s_h.at[idxr.at[k]], bufs[b], sems[b]).wait()
            pltpu.sync_copy(bufs[b], acc.at[idxc.at[k]], add=True)

            @pl.when(k + 4 < CPT)
            def _():
                pltpu.async_copy(s_h.at[idxr.at[k + 4]], bufs[b], sems[b])

        return 0

    lax.fori_loop(0, CPT // 4, _grp, 0)
    plsc.subcore_barrier()

    pbase = cid * HALF + sid * RPT
    for off, nn in ((0, UB), (UB, UB), (2 * UB, RPT - 2 * UB)):
        pltpu.sync_copy(acc.at[pl.ds(abase + off, nn)], g0.at[pl.ds(0, nn)])
        pltpu.sync_copy(s_h.at[pl.ds(pbase + off, nn)], g1.at[pl.ds(0, nn)])
        pltpu.sync_copy(d2_h.at[pl.ds(pbase + off, nn)], g2.at[pl.ds(0, nn)])
        pltpu.sync_copy(zz_h.at[pl.ds(pbase + off, nn)], g3.at[pl.ds(0, nn)])

        def _u(i, _):
            r = i >> 2
            lo = (i & 3) * 16
            a = g0[r, pl.ds(lo, 16)]
            sv = g1[r, pl.ds(lo, 16)]
            dv = g2[r, pl.ds(lo, 16)]
            zv = g3[r, pl.ds(lo, 16)]
            g0[r, pl.ds(lo, 16)] = dv * (a + sv) + zv
            return 0

        lax.fori_loop(0, nn * (C // 16), _u, 0)
        pltpu.sync_copy(g0.at[pl.ds(0, nn)], out_h.at[pl.ds(pbase + off, nn)])


_UPD_CHUNKS = ((0, UB), (UB, UB), (2 * UB, RPT - 2 * UB))


@functools.partial(
    pl.kernel,
    out_type=jax.ShapeDtypeStruct((NPAD, C), jnp.float32),
    mesh=_MESH,
    compiler_params=pltpu.CompilerParams(use_tc_tiling_on_sc=False),
    scratch_types=[
        pltpu.VMEM_SHARED((HALF, C), jnp.float32),
        pltpu.HBM((NPAD, C), jnp.float32),
        pltpu.HBM((2, 16, 16), jnp.int32),
        pltpu.VMEM((CPT, CH), jnp.int32),
        pltpu.VMEM((CPT, CH), jnp.int32),
        pltpu.VMEM((CH, C), jnp.float32),
        pltpu.VMEM((CH, C), jnp.float32),
        pltpu.VMEM((CH, C), jnp.float32),
        pltpu.VMEM((CH, C), jnp.float32),
        pltpu.SemaphoreType.REGULAR,
        pltpu.SemaphoreType.DMA,
        pltpu.SemaphoreType.DMA,
        pltpu.SemaphoreType.DMA,
        pltpu.SemaphoreType.DMA,
    ],
)
def _sc_appnp(s0_h, rowp_h, colf_h, d2_h, zz_h, out_h,
              acc, y_h, flags_h, idxr, idxc, g0, g1, g2, g3,
              rsem, sem0, sem1, sem2, sem3):
    cid = lax.axis_index("c")
    sid = lax.axis_index("s")
    other = 1 - cid
    bufs = (g0, g1, g2, g3)
    sems = (sem0, sem1, sem2, sem3)
    abase = sid * RPT
    pbase = cid * HALF + abase

    pltpu.sync_copy(rowp_h.at[cid, sid], idxr)
    pltpu.sync_copy(colf_h.at[cid, sid], idxc)

    zero16 = jnp.zeros((16,), jnp.float32)

    def _z(i, _):
        g0[i >> 2, pl.ds((i & 3) * 16, 16)] = zero16
        return 0

    lax.fori_loop(0, UB * (C // 16), _z, 0)
    for off, nn in _UPD_CHUNKS:
        pltpu.sync_copy(g0.at[pl.ds(0, nn)], acc.at[pl.ds(abase + off, nn)])

    for off, nn in _UPD_CHUNKS:
        pltpu.sync_copy(s0_h.at[pl.ds(pbase + off, nn)], g1.at[pl.ds(0, nn)])
        pltpu.sync_copy(g1.at[pl.ds(0, nn)], out_h.at[pl.ds(pbase + off, nn)])
    plsc.subcore_barrier()

    def _signal_and_wait(slot):
        del slot

        @pl.when(sid == 0)
        def _():
            pl.semaphore_signal(rsem, 1, core_index=other)
            pl.semaphore_wait(rsem, 1)

        plsc.subcore_barrier()

    _signal_and_wait(0)

    def _round(src, dst, slot):
        for b in range(4):
            pltpu.async_copy(src.at[idxr.at[b]], bufs[b], sems[b])

        def _grp(t, _):
            for b in range(4):
                k = 4 * t + b
                pltpu.make_async_copy(src.at[idxr.at[k]], bufs[b],
                                      sems[b]).wait()
                pltpu.sync_copy(bufs[b], acc.at[idxc.at[k]], add=True)

                @pl.when(k + 4 < CPT)
                def _():
                    pltpu.async_copy(src.at[idxr.at[k + 4]], bufs[b], sems[b])

            return 0

        lax.fori_loop(0, CPT // 4, _grp, 0)
        plsc.subcore_barrier()

        for off, nn in _UPD_CHUNKS:
            pltpu.sync_copy(acc.at[pl.ds(abase + off, nn)],
                            g0.at[pl.ds(0, nn)])
            pltpu.sync_copy(src.at[pl.ds(pbase + off, nn)],
                            g1.at[pl.ds(0, nn)])
            pltpu.sync_copy(d2_h.at[pl.ds(pbase + off, nn)],
                            g2.at[pl.ds(0, nn)])
            pltpu.sync_copy(zz_h.at[pl.ds(pbase + off, nn)],
                            g3.at[pl.ds(0, nn)])

            def _u(i, _):
                r = i >> 2
                lo = (i & 3) * 16
                a = g0[r, pl.ds(lo, 16)]
                sv = g1[r, pl.ds(lo, 16)]
                dv = g2[r, pl.ds(lo, 16)]
                zv = g3[r, pl.ds(lo, 16)]
                g0[r, pl.ds(lo, 16)] = dv * (a + sv) + zv
                g1[r, pl.ds(lo, 16)] = jnp.zeros((16,), jnp.float32)
                return 0

            lax.fori_loop(0, nn * (C // 16), _u, 0)
            pltpu.sync_copy(g0.at[pl.ds(0, nn)],
                            dst.at[pl.ds(pbase + off, nn)])
            pltpu.sync_copy(g1.at[pl.ds(0, nn)],
                            acc.at[pl.ds(abase + off, nn)])
        plsc.subcore_barrier()
        _signal_and_wait(slot)

    for t in range(K // 2):
        _round(out_h, y_h, 2 * t + 1)
        _round(y_h, out_h, 2 * t + 2)


def kernel(x, edge_index, W1, b1, W2, b2):
    row2d = edge_index[0].reshape(E // 128, 128)
    col2d = edge_index[1].reshape(E // 128, 128)
    rowm, c0, c1 = _idx_prep(row2d, col2d)

    rowm_e = jnp.concatenate([rowm.reshape(-1),
                              jnp.full((1,), NH, jnp.int32)])
    c0_e = jnp.concatenate([c0.reshape(-1), jnp.full((1,), NH, jnp.int32)])
    c1_e = jnp.concatenate([c1.reshape(-1), jnp.full((1,), NH, jnp.int32)])
    key0 = edge_index[1] < NH
    pos0 = jnp.cumsum(key0.astype(jnp.int32)) - 1
    pos1 = jnp.cumsum(1 - key0.astype(jnp.int32)) - 1
    idxe = jnp.arange(E, dtype=jnp.int32)
    oob = jnp.int32(EHP)
    list0 = jnp.full((EHP,), E, jnp.int32).at[
        jnp.where(key0, pos0, oob)].set(idxe, mode="drop")
    list1 = jnp.full((EHP,), E, jnp.int32).at[
        jnp.where(key0, oob, pos1)].set(idxe, mode="drop")
    rowp = jnp.stack([rowm_e[list0], rowm_e[list1]]).reshape(2, 16, CPT, CH)
    colf = jnp.stack([c0_e[list0], c1_e[list1]]).reshape(2, 16, CPT, CH)

    h = _mlp(x, W1, b1, W2, b2)
    zpad = jnp.zeros((HALF - NH, C), jnp.float32)
    zp = jnp.concatenate([h[:NH], zpad, h[NH:], zpad])

    ones = jnp.ones((NPAD, C), jnp.float32)
    zeros = jnp.zeros((NPAD, C), jnp.float32)
    sdeg = _sc_round(ones, rowp, colf, ones, zeros)

    s0, D2, ZZ = _prep(sdeg, zp)
    s = _sc_appnp(s0, rowp, colf, D2, ZZ)
    o = _final(s, sdeg)
    return jnp.concatenate([o[:NH], o[HALF:HALF + NH]])


# speedup vs baseline: 1.1422x; 1.1097x over previous
import functools

import jax
import jax.numpy as jnp
from jax import lax
from jax.experimental import pallas as pl
from jax.experimental.pallas import tpu as pltpu
from jax.experimental.pallas import tpu_sc as plsc

N = 10000
E = 320000
D_IN = 128
D_HID = 128
C = 64
K = 10
ALPHA = 0.1

NH = 5000
HALF = 5120
NPAD = 2 * HALF
RPT = HALF // 16
CH = 256
CPT = 80
UB = 128
TPE = CH * CPT
EPADT = 16 * TPE


def _idx_prep_body(row_ref, col_ref, rowm_ref, c0_ref, c1_ref):
    r = row_ref[...]
    c = col_ref[...]
    io8 = lax.broadcasted_iota(jnp.int32, r.shape, 1) & 7
    dummy = NH + io8
    rowm_ref[...] = jnp.where(r >= NH, r + (HALF - NH), r)
    c0_ref[...] = jnp.where(c < NH, c, dummy)
    c1_ref[...] = jnp.where(c >= NH, c - NH, dummy)


def _idx_prep(row2d, col2d):
    shp = jax.ShapeDtypeStruct(row2d.shape, jnp.int32)
    return pl.pallas_call(
        _idx_prep_body,
        out_shape=[shp, shp, shp],
    )(row2d, col2d)


def _mlp_body(x_ref, w1_ref, b1_ref, w2_ref, b2_ref, o_ref):
    h = jnp.dot(x_ref[...], w1_ref[...], preferred_element_type=jnp.float32)
    h = jnp.maximum(h + b1_ref[...], 0.0)
    o = jnp.dot(h, w2_ref[...], preferred_element_type=jnp.float32)
    o_ref[...] = o + b2_ref[...]


def _mlp(x, W1, b1, W2, b2):
    return pl.pallas_call(
        _mlp_body,
        grid=(10,),
        in_specs=[
            pl.BlockSpec((N // 10, D_IN), lambda i: (i, 0)),
            pl.BlockSpec((D_IN, D_HID), lambda i: (0, 0)),
            pl.BlockSpec((1, D_HID), lambda i: (0, 0)),
            pl.BlockSpec((D_HID, C), lambda i: (0, 0)),
            pl.BlockSpec((1, C), lambda i: (0, 0)),
        ],
        out_specs=pl.BlockSpec((N // 10, C), lambda i: (i, 0)),
        out_shape=jax.ShapeDtypeStruct((N, C), jnp.float32),
    )(x, W1, b1.reshape(1, D_HID), W2, b2.reshape(1, C))


def _prep_body(sdeg_ref, z_ref, s0_ref, d2_ref, zz_ref):
    deg = sdeg_ref[...]
    z = z_ref[...]
    rows = lax.broadcasted_iota(jnp.int32, deg.shape, 0)
    m = rows < NH
    dinv = jnp.where(m, lax.rsqrt(jnp.maximum(deg, 1e-12)), 0.0)
    s0_ref[...] = dinv * z
    d2_ref[...] = (1.0 - ALPHA) * dinv * dinv
    zz_ref[...] = ALPHA * dinv * z


def _prep(sdeg, zp):
    shp = jax.ShapeDtypeStruct((NPAD, C), jnp.float32)
    spec = pl.BlockSpec((HALF, C), lambda i: (i, 0))
    return pl.pallas_call(
        _prep_body,
        grid=(2,),
        in_specs=[spec, spec],
        out_specs=[spec, spec, spec],
        out_shape=[shp, shp, shp],
    )(sdeg, zp)


def _final_body(s_ref, sdeg_ref, o_ref):
    deg = jnp.maximum(sdeg_ref[...], 1e-12)
    o = s_ref[...] * jnp.sqrt(deg)
    mx = jnp.max(o, axis=1, keepdims=True)
    e = jnp.exp(o - mx)
    lse = jnp.log(jnp.sum(e, axis=1, keepdims=True))
    o_ref[...] = (o - mx) - lse


def _final(s, sdeg):
    spec = pl.BlockSpec((HALF, C), lambda i: (i, 0))
    return pl.pallas_call(
        _final_body,
        grid=(2,),
        in_specs=[spec, spec],
        out_specs=spec,
        out_shape=jax.ShapeDtypeStruct((NPAD, C), jnp.float32),
    )(s, sdeg)


_MESH = plsc.VectorSubcoreMesh(core_axis_name="c", subcore_axis_name="s")


@functools.partial(
    pl.kernel,
    out_type=jax.ShapeDtypeStruct((NPAD, C), jnp.float32),
    mesh=_MESH,
    compiler_params=pltpu.CompilerParams(use_tc_tiling_on_sc=False),
    scratch_types=[
        pltpu.VMEM_SHARED((HALF, C), jnp.float32),
        pltpu.VMEM((CPT, CH), jnp.int32),
        pltpu.VMEM((CPT, CH), jnp.int32),
        pltpu.VMEM((CH, C), jnp.float32),
        pltpu.VMEM((CH, C), jnp.float32),
        pltpu.VMEM((CH, C), jnp.float32),
        pltpu.VMEM((CH, C), jnp.float32),
        pltpu.SemaphoreType.DMA,
        pltpu.SemaphoreType.DMA,
        pltpu.SemaphoreType.DMA,
        pltpu.SemaphoreType.DMA,
    ],
)
def _sc_round(s_h, rowp_h, colf_h, d2_h, zz_h, out_h,
              acc, idxr, idxc, g0, g1, g2, g3, sem0, sem1, sem2, sem3):
    cid = lax.axis_index("c")
    sid = lax.axis_index("s")
    gb = (g0, g1, g2, g3)

    pltpu.sync_copy(rowp_h.at[sid], idxr)
    pltpu.sync_copy(colf_h.at[cid, sid], idxc)

    zero16 = jnp.zeros((16,), jnp.float32)

    def _z(i, _):
        g0[i >> 2, pl.ds((i & 3) * 16, 16)] = zero16
        return 0

    lax.fori_loop(0, UB * (C // 16), _z, 0)
    abase = sid * RPT
    pltpu.sync_copy(g0.at[pl.ds(0, UB)], acc.at[pl.ds(abase, UB)])
    pltpu.sync_copy(g0.at[pl.ds(0, UB)], acc.at[pl.ds(abase + UB, UB)])
    pltpu.sync_copy(g0.at[pl.ds(0, RPT - 2 * UB)],
                    acc.at[pl.ds(abase + 2 * UB, RPT - 2 * UB)])
    plsc.subcore_barrier()

    bufs = (g0, g1, g2, g3)
    sems = (sem0, sem1, sem2, sem3)
    for b in range(4):
        pltpu.async_copy(s_h.at[idxr.at[b]], bufs[b], sems[b])

    def _grp(t, _):
        for b in range(4):
            k = 4 * t + b
            pltpu.make_async_copy(s_h.at[idxr.at[k]], bufs[b], sems[b]).wait()
            pltpu.sync_copy(bufs[b], acc.at[idxc.at[k]], add=True)

            @pl.when(k + 4 < CPT)
            def _():
                pltpu.async_copy(s_h.at[idxr.at[k + 4]], bufs[b], sems[b])

        return 0

    lax.fori_loop(0, CPT // 4, _grp, 0)
    plsc.subcore_barrier()

    pbase = cid * HALF + sid * RPT
    for off, nn in ((0, UB), (UB, UB), (2 * UB, RPT - 2 * UB)):
        pltpu.sync_copy(acc.at[pl.ds(abase + off, nn)], g0.at[pl.ds(0, nn)])
        pltpu.sync_copy(s_h.at[pl.ds(pbase + off, nn)], g1.at[pl.ds(0, nn)])
        pltpu.sync_copy(d2_h.at[pl.ds(pbase + off, nn)], g2.at[pl.ds(0, nn)])
        pltpu.sync_copy(zz_h.at[pl.ds(pbase + off, nn)], g3.at[pl.ds(0, nn)])

        def _u(i, _):
            r = i >> 2
            lo = (i & 3) * 16
            a = g0[r, pl.ds(lo, 16)]
            sv = g1[r, pl.ds(lo, 16)]
            dv = g2[r, pl.ds(lo, 16)]
            zv = g3[r, pl.ds(lo, 16)]
            g0[r, pl.ds(lo, 16)] = dv * (a + sv) + zv
            return 0

        lax.fori_loop(0, nn * (C // 16), _u, 0)
        pltpu.sync_copy(g0.at[pl.ds(0, nn)], out_h.at[pl.ds(pbase + off, nn)])


def kernel(x, edge_index, W1, b1, W2, b2):
    row2d = edge_index[0].reshape(E // 128, 128)
    col2d = edge_index[1].reshape(E // 128, 128)
    rowm, c0, c1 = _idx_prep(row2d, col2d)

    pad_n = EPADT - E
    spread = NH + (jnp.arange(pad_n, dtype=jnp.int32) & 7)
    rowp = jnp.concatenate(
        [rowm.reshape(-1), jnp.full((pad_n,), NH, jnp.int32)]
    ).reshape(16, CPT, CH)
    colf = jnp.stack([
        jnp.concatenate([c0.reshape(-1), spread]),
        jnp.concatenate([c1.reshape(-1), spread]),
    ]).reshape(2, 16, CPT, CH)

    h = _mlp(x, W1, b1, W2, b2)
    zpad = jnp.zeros((HALF - NH, C), jnp.float32)
    zp = jnp.concatenate([h[:NH], zpad, h[NH:], zpad])

    ones = jnp.ones((NPAD, C), jnp.float32)
    zeros = jnp.zeros((NPAD, C), jnp.float32)
    sdeg = _sc_round(ones, rowp, colf, ones, zeros)

    s, D2, ZZ = _prep(sdeg, zp)
    for _ in range(K):
        s = _sc_round(s, rowp, colf, D2, ZZ)
    o = _final(s, sdeg)
    return jnp.concatenate([o[:NH], o[HALF:HALF + NH]])
